# baseline (speedup 1.0000x reference)
import jax
import jax.numpy as jnp
from jax import lax
from jax.experimental import pallas as pl
from jax.experimental.pallas import tpu as pltpu
from jax.experimental.pallas import tpu_sc as plsc

N_OUT = 768
C_IN = 768
HW = 9
M = 4
TOTAL = N_OUT * C_IN * HW
NWORKERS = 32
PER_TILE = TOTAL // NWORKERS
GBLK = M * HW
UNIT = 16 * GBLK
CHUNK_UNITS = 36
CHUNK = CHUNK_UNITS * UNIT
NCHUNK = PER_TILE // CHUNK
NPAIR = NCHUNK // 2


def _body(w_hbm, out_hbm, bin0, bin1, bout0, bout1, si0, si1, so0, so1):
    cid = lax.axis_index("c")
    sid = lax.axis_index("s")
    wid = sid * 2 + cid
    tbase = wid * PER_TILE
    bins = (bin0, bin1)
    bouts = (bout0, bout1)
    sis = (si0, si1)
    sos = (so0, so1)

    lane36 = lax.iota(jnp.int32, 16) * GBLK

    pltpu.async_copy(w_hbm.at[pl.ds(tbase, CHUNK)], bin0, si0)
    pltpu.async_copy(w_hbm.at[pl.ds(tbase + CHUNK, CHUNK)], bin1, si1)

    def compute_chunk(bin_ref, bout_ref):
        def unit_body(u, carry):
            base = lane36 + u * UNIT
            for j in range(HW):
                idx = [base + (9 * k + j) for k in range(M)]
                a = [plsc.load_gather(bin_ref, [idx[k]]) for k in range(M)]
                n = [jnp.abs(a[k]) + jnp.float32(1e-7) for k in range(M)]
                c01 = n[0] >= n[1]
                c02 = n[0] >= n[2]
                c03 = n[0] >= n[3]
                c12 = n[1] >= n[2]
                c13 = n[1] >= n[3]
                c23 = n[2] >= n[3]
                k0 = (c01 & (c02 | c03)) | (c02 & c03)
                k1 = (~c01 & (c12 | c13)) | (c12 & c13)
                k2 = (~c02 & (~c12 | c23)) | (~c12 & c23)
                k3 = ~((c03 & (c13 | c23)) | (c13 & c23))
                keep = (k0, k1, k2, k3)
                zero = jnp.zeros((16,), jnp.float32)
                for k in range(M):
                    plsc.store_scatter(bout_ref, [idx[k]],
                                       jnp.where(keep[k], a[k], zero))
            return carry

        lax.fori_loop(0, CHUNK_UNITS, unit_body, 0)

    def pair_body(g2, carry):
        for b in range(2):
            off = tbase + (g2 * 2 + b) * CHUNK
            pltpu.make_async_copy(
                w_hbm.at[pl.ds(tbase, CHUNK)], bins[b], sis[b]).wait()

            @pl.when(g2 > 0)
            def _():
                pltpu.make_async_copy(
                    bouts[b], out_hbm.at[pl.ds(tbase, CHUNK)], sos[b]).wait()

            compute_chunk(bins[b], bouts[b])

            pltpu.async_copy(bouts[b], out_hbm.at[pl.ds(off, CHUNK)], sos[b])

            @pl.when(g2 < NPAIR - 1)
            def _():
                pltpu.async_copy(
                    w_hbm.at[pl.ds(off + 2 * CHUNK, CHUNK)], bins[b], sis[b])
        return carry

    lax.fori_loop(0, NPAIR, pair_body, 0)

    for b in range(2):
        pltpu.make_async_copy(
            bouts[b], out_hbm.at[pl.ds(tbase, CHUNK)], sos[b]).wait()


def kernel(weight):
    wf = weight.reshape(TOTAL)
    mesh = plsc.VectorSubcoreMesh(core_axis_name="c", subcore_axis_name="s")
    f = pl.kernel(
        _body,
        out_type=jax.ShapeDtypeStruct((TOTAL,), jnp.float32),
        mesh=mesh,
        compiler_params=pltpu.CompilerParams(needs_layout_passes=False),
        scratch_types=[
            pltpu.VMEM((CHUNK,), jnp.float32),
            pltpu.VMEM((CHUNK,), jnp.float32),
            pltpu.VMEM((CHUNK,), jnp.float32),
            pltpu.VMEM((CHUNK,), jnp.float32),
            pltpu.SemaphoreType.DMA,
            pltpu.SemaphoreType.DMA,
            pltpu.SemaphoreType.DMA,
            pltpu.SemaphoreType.DMA,
        ],
    )
    return f(wf).reshape(weight.shape)


# speedup vs baseline: 119.3151x; 119.3151x over previous
import jax
import jax.numpy as jnp
from jax import lax
from jax.experimental import pallas as pl
from jax.experimental.pallas import tpu as pltpu
from jax.experimental.pallas import tpu_sc as plsc

N_OUT = 768
C_IN = 768
HW = 9
M = 4
TOTAL = N_OUT * C_IN * HW
NWORKERS = 32
PER_TILE = TOTAL // NWORKERS
GBLK = M * HW
UNIT = 16 * GBLK
CHUNK_UNITS = 36
CHUNK = CHUNK_UNITS * UNIT
NCHUNK = PER_TILE // CHUNK
NPAIR = NCHUNK // 2


def _body(w_hbm, out_hbm, bin0, bin1, bout0, bout1, si0, si1, so0, so1):
    cid = lax.axis_index("c")
    sid = lax.axis_index("s")
    wid = sid * 2 + cid
    tbase = wid * PER_TILE
    bins = (bin0, bin1)
    bouts = (bout0, bout1)
    sis = (si0, si1)
    sos = (so0, so1)

    lane4 = lax.iota(jnp.int32, 16) * M

    pltpu.async_copy(w_hbm.at[pl.ds(tbase, CHUNK)], bin0, si0)
    pltpu.async_copy(w_hbm.at[pl.ds(tbase + CHUNK, CHUNK)], bin1, si1)

    def compute_chunk(bin_ref, bout_ref):
        def unit_body(u, carry):
            base = lane4 + u * UNIT
            for j in range(HW):
                idx = [base + (64 * j + k) for k in range(M)]
                a = [plsc.load_gather(bin_ref, [idx[k]]) for k in range(M)]
                n = [jnp.abs(a[k]) + jnp.float32(1e-7) for k in range(M)]
                c01 = n[0] >= n[1]
                c02 = n[0] >= n[2]
                c03 = n[0] >= n[3]
                c12 = n[1] >= n[2]
                c13 = n[1] >= n[3]
                c23 = n[2] >= n[3]
                k0 = (c01 & (c02 | c03)) | (c02 & c03)
                k1 = (~c01 & (c12 | c13)) | (c12 & c13)
                k2 = (~c02 & (~c12 | c23)) | (~c12 & c23)
                k3 = ~((c03 & (c13 | c23)) | (c13 & c23))
                keep = (k0, k1, k2, k3)
                zero = jnp.zeros((16,), jnp.float32)
                for k in range(M):
                    plsc.store_scatter(bout_ref, [idx[k]],
                                       jnp.where(keep[k], a[k], zero))
            return carry

        lax.fori_loop(0, CHUNK_UNITS, unit_body, 0)

    def pair_body(g2, carry):
        for b in range(2):
            off = tbase + (g2 * 2 + b) * CHUNK
            pltpu.make_async_copy(
                w_hbm.at[pl.ds(tbase, CHUNK)], bins[b], sis[b]).wait()

            @pl.when(g2 > 0)
            def _():
                pltpu.make_async_copy(
                    bouts[b], out_hbm.at[pl.ds(tbase, CHUNK)], sos[b]).wait()

            compute_chunk(bins[b], bouts[b])

            pltpu.async_copy(bouts[b], out_hbm.at[pl.ds(off, CHUNK)], sos[b])

            @pl.when(g2 < NPAIR - 1)
            def _():
                pltpu.async_copy(
                    w_hbm.at[pl.ds(off + 2 * CHUNK, CHUNK)], bins[b], sis[b])
        return carry

    lax.fori_loop(0, NPAIR, pair_body, 0)

    for b in range(2):
        pltpu.make_async_copy(
            bouts[b], out_hbm.at[pl.ds(tbase, CHUNK)], sos[b]).wait()


def kernel(weight):
    wf = (weight.transpose(2, 3, 0, 1)
          .reshape(3, 3, 96, 8, 6, 128)
          .transpose(0, 1, 2, 4, 3, 5)
          .reshape(TOTAL))
    mesh = plsc.VectorSubcoreMesh(core_axis_name="c", subcore_axis_name="s")
    f = pl.kernel(
        _body,
        out_type=jax.ShapeDtypeStruct((TOTAL,), jnp.float32),
        mesh=mesh,
        compiler_params=pltpu.CompilerParams(needs_layout_passes=False),
        scratch_types=[
            pltpu.VMEM((CHUNK,), jnp.float32),
            pltpu.VMEM((CHUNK,), jnp.float32),
            pltpu.VMEM((CHUNK,), jnp.float32),
            pltpu.VMEM((CHUNK,), jnp.float32),
            pltpu.SemaphoreType.DMA,
            pltpu.SemaphoreType.DMA,
            pltpu.SemaphoreType.DMA,
            pltpu.SemaphoreType.DMA,
        ],
    )
    of = f(wf)
    return (of.reshape(3, 3, 96, 6, 8, 128)
            .transpose(0, 1, 2, 4, 3, 5)
            .reshape(3, 3, 768, 768)
            .transpose(2, 3, 0, 1))


# speedup vs baseline: 150.7003x; 1.2630x over previous
import jax
import jax.numpy as jnp
from jax import lax
from jax.experimental import pallas as pl
from jax.experimental.pallas import tpu as pltpu
from jax.experimental.pallas import tpu_sc as plsc

N_OUT = 768
C_IN = 768
HW = 9
M = 4
TOTAL = N_OUT * C_IN * HW
NWORKERS = 32
PER_TILE = TOTAL // NWORKERS
GBLK = M * HW
UNIT = 16 * GBLK
CHUNK_UNITS = 36
CHUNK = CHUNK_UNITS * UNIT
NCHUNK = PER_TILE // CHUNK
NPAIR = NCHUNK // 2


def _body(w_hbm, out_hbm, bin0, bin1, bout0, bout1, si0, si1, so0, so1):
    cid = lax.axis_index("c")
    sid = lax.axis_index("s")
    wid = sid * 2 + cid
    tbase = wid * PER_TILE
    bins = (bin0, bin1)
    bouts = (bout0, bout1)
    sis = (si0, si1)
    sos = (so0, so1)

    lane4 = lax.iota(jnp.int32, 16) * M

    pltpu.async_copy(w_hbm.at[pl.ds(tbase, CHUNK)], bin0, si0)
    pltpu.async_copy(w_hbm.at[pl.ds(tbase + CHUNK, CHUNK)], bin1, si1)

    def compute_chunk(bin_ref, bout_ref):
        @plsc.parallel_loop(0, CHUNK_UNITS)
        def unit_body(u):
            for j in range(HW):
                src = bin_ref.at[pl.ds(u * UNIT + 64 * j, 64)]
                dst = bout_ref.at[pl.ds(u * UNIT + 64 * j, 64)]
                idx = [lane4 + k for k in range(M)]
                a = [plsc.load_gather(src, [idx[k]]) for k in range(M)]
                n = [jnp.abs(a[k]) + jnp.float32(1e-7) for k in range(M)]
                c01 = n[0] >= n[1]
                c02 = n[0] >= n[2]
                c03 = n[0] >= n[3]
                c12 = n[1] >= n[2]
                c13 = n[1] >= n[3]
                c23 = n[2] >= n[3]
                k0 = (c01 & (c02 | c03)) | (c02 & c03)
                k1 = (~c01 & (c12 | c13)) | (c12 & c13)
                k2 = (~c02 & (~c12 | c23)) | (~c12 & c23)
                k3 = ~((c03 & (c13 | c23)) | (c13 & c23))
                keep = (k0, k1, k2, k3)
                zero = jnp.zeros((16,), jnp.float32)
                for k in range(M):
                    plsc.store_scatter(dst, [idx[k]],
                                       jnp.where(keep[k], a[k], zero))

    def pair_body(g2, carry):
        for b in range(2):
            off = tbase + (g2 * 2 + b) * CHUNK
            pltpu.make_async_copy(
                w_hbm.at[pl.ds(tbase, CHUNK)], bins[b], sis[b]).wait()

            @pl.when(g2 > 0)
            def _():
                pltpu.make_async_copy(
                    bouts[b], out_hbm.at[pl.ds(tbase, CHUNK)], sos[b]).wait()

            compute_chunk(bins[b], bouts[b])

            pltpu.async_copy(bouts[b], out_hbm.at[pl.ds(off, CHUNK)], sos[b])

            @pl.when(g2 < NPAIR - 1)
            def _():
                pltpu.async_copy(
                    w_hbm.at[pl.ds(off + 2 * CHUNK, CHUNK)], bins[b], sis[b])
        return carry

    lax.fori_loop(0, NPAIR, pair_body, 0)

    for b in range(2):
        pltpu.make_async_copy(
            bouts[b], out_hbm.at[pl.ds(tbase, CHUNK)], sos[b]).wait()


def kernel(weight):
    wf = (weight.transpose(2, 3, 0, 1)
          .reshape(3, 3, 96, 8, 6, 128)
          .transpose(0, 1, 2, 4, 3, 5)
          .reshape(TOTAL))
    mesh = plsc.VectorSubcoreMesh(core_axis_name="c", subcore_axis_name="s")
    f = pl.kernel(
        _body,
        out_type=jax.ShapeDtypeStruct((TOTAL,), jnp.float32),
        mesh=mesh,
        compiler_params=pltpu.CompilerParams(needs_layout_passes=False),
        scratch_types=[
            pltpu.VMEM((CHUNK,), jnp.float32),
            pltpu.VMEM((CHUNK,), jnp.float32),
            pltpu.VMEM((CHUNK,), jnp.float32),
            pltpu.VMEM((CHUNK,), jnp.float32),
            pltpu.SemaphoreType.DMA,
            pltpu.SemaphoreType.DMA,
            pltpu.SemaphoreType.DMA,
            pltpu.SemaphoreType.DMA,
        ],
    )
    of = f(wf)
    return (of.reshape(3, 3, 96, 6, 8, 128)
            .transpose(0, 1, 2, 4, 3, 5)
            .reshape(3, 3, 768, 768)
            .transpose(2, 3, 0, 1))


# speedup vs baseline: 151.4815x; 1.0052x over previous
import jax
import jax.numpy as jnp
from jax import lax
from jax.experimental import pallas as pl
from jax.experimental.pallas import tpu as pltpu
from jax.experimental.pallas import tpu_sc as plsc

N_OUT = 768
C_IN = 768
HW = 9
M = 4
TOTAL = N_OUT * C_IN * HW
NWORKERS = 32
PER_TILE = TOTAL // NWORKERS
GBLK = M * HW
UNIT = 16 * GBLK
CHUNK_UNITS = 36
CHUNK = CHUNK_UNITS * UNIT
NCHUNK = PER_TILE // CHUNK
NPAIR = NCHUNK // 2


def _body(w_hbm, out_hbm, bin0, bin1, bout0, bout1, si0, si1, so0, so1):
    cid = lax.axis_index("c")
    sid = lax.axis_index("s")
    wid = sid * 2 + cid
    tbase = wid * PER_TILE
    bins = (bin0, bin1)
    bouts = (bout0, bout1)
    sis = (si0, si1)
    sos = (so0, so1)

    lane4 = lax.iota(jnp.int32, 16) * M

    pltpu.async_copy(w_hbm.at[pl.ds(tbase, CHUNK)], bin0, si0)
    pltpu.async_copy(w_hbm.at[pl.ds(tbase + CHUNK, CHUNK)], bin1, si1)

    def compute_chunk(bin_ref, bout_ref):
        @plsc.parallel_loop(0, CHUNK_UNITS)
        def unit_body(u):
            for j in range(HW):
                src = bin_ref.at[pl.ds(u * UNIT + 64 * j, 64)]
                dst = bout_ref.at[pl.ds(u * UNIT + 64 * j, 64)]
                idx = [lane4 + k for k in range(M)]
                a = [plsc.load_gather(src, [idx[k]]) for k in range(M)]
                n = [jnp.abs(a[k]) + jnp.float32(1e-7) for k in range(M)]
                c01 = n[0] >= n[1]
                c02 = n[0] >= n[2]
                c03 = n[0] >= n[3]
                c12 = n[1] >= n[2]
                c13 = n[1] >= n[3]
                c23 = n[2] >= n[3]
                k0 = (c01 & (c02 | c03)) | (c02 & c03)
                k1 = (~c01 & (c12 | c13)) | (c12 & c13)
                k2 = (~c02 & (~c12 | c23)) | (~c12 & c23)
                k3 = k0 ^ k1 ^ k2
                keep = (k0, k1, k2, k3)
                zero = jnp.zeros((16,), jnp.float32)
                for k in range(M):
                    plsc.store_scatter(dst, [idx[k]],
                                       jnp.where(keep[k], a[k], zero))

    def pair_body(g2, carry):
        for b in range(2):
            off = tbase + (g2 * 2 + b) * CHUNK
            pltpu.make_async_copy(
                w_hbm.at[pl.ds(tbase, CHUNK)], bins[b], sis[b]).wait()

            @pl.when(g2 > 0)
            def _():
                pltpu.make_async_copy(
                    bouts[b], out_hbm.at[pl.ds(tbase, CHUNK)], sos[b]).wait()

            compute_chunk(bins[b], bouts[b])

            pltpu.async_copy(bouts[b], out_hbm.at[pl.ds(off, CHUNK)], sos[b])

            @pl.when(g2 < NPAIR - 1)
            def _():
                pltpu.async_copy(
                    w_hbm.at[pl.ds(off + 2 * CHUNK, CHUNK)], bins[b], sis[b])
        return carry

    lax.fori_loop(0, NPAIR, pair_body, 0)

    for b in range(2):
        pltpu.make_async_copy(
            bouts[b], out_hbm.at[pl.ds(tbase, CHUNK)], sos[b]).wait()


def kernel(weight):
    wf = (weight.transpose(2, 3, 0, 1)
          .reshape(3, 3, 96, 8, 6, 128)
          .transpose(0, 1, 2, 4, 3, 5)
          .reshape(TOTAL))
    mesh = plsc.VectorSubcoreMesh(core_axis_name="c", subcore_axis_name="s")
    f = pl.kernel(
        _body,
        out_type=jax.ShapeDtypeStruct((TOTAL,), jnp.float32),
        mesh=mesh,
        compiler_params=pltpu.CompilerParams(
            needs_layout_passes=False, skip_device_barrier=True),
        scratch_types=[
            pltpu.VMEM((CHUNK,), jnp.float32),
            pltpu.VMEM((CHUNK,), jnp.float32),
            pltpu.VMEM((CHUNK,), jnp.float32),
            pltpu.VMEM((CHUNK,), jnp.float32),
            pltpu.SemaphoreType.DMA,
            pltpu.SemaphoreType.DMA,
            pltpu.SemaphoreType.DMA,
            pltpu.SemaphoreType.DMA,
        ],
    )
    of = f(wf)
    return (of.reshape(3, 3, 96, 6, 8, 128)
            .transpose(0, 1, 2, 4, 3, 5)
            .reshape(3, 3, 768, 768)
            .transpose(2, 3, 0, 1))


# speedup vs baseline: 153.9138x; 1.0161x over previous
import jax
import jax.numpy as jnp
from jax import lax
from jax.experimental import pallas as pl
from jax.experimental.pallas import tpu as pltpu
from jax.experimental.pallas import tpu_sc as plsc

N_OUT = 768
C_IN = 768
HW = 9
M = 4
TOTAL = N_OUT * C_IN * HW
NWORKERS = 32
PER_TILE = TOTAL // NWORKERS
GBLK = M * HW
UNIT = 16 * GBLK
CHUNK_UNITS = 36
CHUNK = CHUNK_UNITS * UNIT
NCHUNK = PER_TILE // CHUNK
NPAIR = NCHUNK // 2


def _body(w_hbm, out_hbm, bin0, bin1, bout0, bout1, si0, si1, so0, so1):
    cid = lax.axis_index("c")
    sid = lax.axis_index("s")
    wid = sid * 2 + cid
    tbase = wid * PER_TILE
    bins = (bin0, bin1)
    bouts = (bout0, bout1)
    sis = (si0, si1)
    sos = (so0, so1)

    lane4 = lax.iota(jnp.int32, 16) * M
    ionev = jnp.ones((16,), jnp.int32)
    izerov = jnp.zeros((16,), jnp.int32)

    pltpu.async_copy(w_hbm.at[pl.ds(tbase, CHUNK)], bin0, si0)
    pltpu.async_copy(w_hbm.at[pl.ds(tbase + CHUNK, CHUNK)], bin1, si1)

    def compute_chunk(bin_ref, bout_ref):
        @plsc.parallel_loop(0, CHUNK_UNITS)
        def unit_body(u):
            for j in range(HW):
                src = bin_ref.at[pl.ds(u * UNIT + 64 * j, 64)]
                dst = bout_ref.at[pl.ds(u * UNIT + 64 * j, 64)]
                idx = [lane4 + k for k in range(M)]
                a = [plsc.load_gather(src, [idx[k]]) for k in range(M)]
                n = [jnp.abs(a[k]) + jnp.float32(1e-7) for k in range(M)]
                c01 = n[0] >= n[1]
                c02 = n[0] >= n[2]
                c03 = n[0] >= n[3]
                c12 = n[1] >= n[2]
                c13 = n[1] >= n[3]
                c23 = n[2] >= n[3]
                x01 = jnp.where(c01, ionev, izerov)
                x02 = jnp.where(c02, ionev, izerov)
                x03 = jnp.where(c03, ionev, izerov)
                x12 = jnp.where(c12, ionev, izerov)
                x13 = jnp.where(c13, ionev, izerov)
                x23 = jnp.where(c23, ionev, izerov)
                k0 = (x01 + x02 + x03) >= 2
                k1 = (x12 + x13 - x01) >= 1
                k2 = x23 >= (x02 + x12)
                k3 = k0 ^ k1 ^ k2
                keep = (k0, k1, k2, k3)
                zero = jnp.zeros((16,), jnp.float32)
                for k in range(M):
                    plsc.store_scatter(dst, [idx[k]],
                                       jnp.where(keep[k], a[k], zero))

    def pair_body(g2, carry):
        for b in range(2):
            off = tbase + (g2 * 2 + b) * CHUNK
            pltpu.make_async_copy(
                w_hbm.at[pl.ds(tbase, CHUNK)], bins[b], sis[b]).wait()

            @pl.when(g2 > 0)
            def _():
                pltpu.make_async_copy(
                    bouts[b], out_hbm.at[pl.ds(tbase, CHUNK)], sos[b]).wait()

            compute_chunk(bins[b], bouts[b])

            pltpu.async_copy(bouts[b], out_hbm.at[pl.ds(off, CHUNK)], sos[b])

            @pl.when(g2 < NPAIR - 1)
            def _():
                pltpu.async_copy(
                    w_hbm.at[pl.ds(off + 2 * CHUNK, CHUNK)], bins[b], sis[b])
        return carry

    lax.fori_loop(0, NPAIR, pair_body, 0)

    for b in range(2):
        pltpu.make_async_copy(
            bouts[b], out_hbm.at[pl.ds(tbase, CHUNK)], sos[b]).wait()


def kernel(weight):
    wf = (weight.transpose(2, 3, 0, 1)
          .reshape(3, 3, 96, 8, 6, 128)
          .transpose(0, 1, 2, 4, 3, 5)
          .reshape(TOTAL))
    mesh = plsc.VectorSubcoreMesh(core_axis_name="c", subcore_axis_name="s")
    f = pl.kernel(
        _body,
        out_type=jax.ShapeDtypeStruct((TOTAL,), jnp.float32),
        mesh=mesh,
        compiler_params=pltpu.CompilerParams(
            needs_layout_passes=False, skip_device_barrier=True),
        scratch_types=[
            pltpu.VMEM((CHUNK,), jnp.float32),
            pltpu.VMEM((CHUNK,), jnp.float32),
            pltpu.VMEM((CHUNK,), jnp.float32),
            pltpu.VMEM((CHUNK,), jnp.float32),
            pltpu.SemaphoreType.DMA,
            pltpu.SemaphoreType.DMA,
            pltpu.SemaphoreType.DMA,
            pltpu.SemaphoreType.DMA,
        ],
    )
    of = f(wf)
    return (of.reshape(3, 3, 96, 6, 8, 128)
            .transpose(0, 1, 2, 4, 3, 5)
            .reshape(3, 3, 768, 768)
            .transpose(2, 3, 0, 1))
